# baseline (speedup 1.0000x reference)
import jax
import jax.numpy as jnp
from jax.experimental import pallas as pl
from jax.experimental.pallas import tpu as pltpu


def _band_t(conv_weight: jnp.ndarray, channels: int, scale: float):
    k = conv_weight.shape[0]
    pad = (k - 1) // 2
    c = jnp.arange(channels)
    tap = c[:, None] - c[None, :] + pad
    valid = (tap >= 0) & (tap < k)
    w = jnp.where(valid, jnp.take(conv_weight, jnp.clip(tap, 0, k - 1)), 0.0)
    return (w * scale).astype(jnp.float32)


def _eca_body(x_ref, bt_ref, o_ref):
    x = x_ref[...]
    s = jnp.sum(x, axis=0)
    conv = jnp.dot(s, bt_ref[...],
                   preferred_element_type=jnp.float32)
    att = 1.0 / (1.0 + jnp.exp(-conv))
    o_ref[...] = x * att[None].astype(x.dtype)


def kernel(x, conv_weight):
    N, C, H, W = x.shape
    HW = H * W
    xt = jnp.transpose(x, (2, 3, 0, 1)).reshape(HW, N, C)
    bt = _band_t(conv_weight, C, 1.0 / float(HW))

    Bn = 32 if N % 32 == 0 else N

    out = pl.pallas_call(
        _eca_body,
        out_shape=jax.ShapeDtypeStruct((HW, N, C), x.dtype),
        grid_spec=pl.GridSpec(
            grid=(N // Bn,),
            in_specs=[pl.BlockSpec((HW, Bn, C), lambda i: (0, i, 0)),
                      pl.BlockSpec((C, C), lambda i: (0, 0))],
            out_specs=pl.BlockSpec((HW, Bn, C), lambda i: (0, i, 0)),
        ),
        compiler_params=pltpu.CompilerParams(
            dimension_semantics=("parallel",),
            vmem_limit_bytes=58 << 20,
        ),
        cost_estimate=pl.CostEstimate(
            flops=2 * N * C * HW + 2 * N * C * C,
            transcendentals=N * C,
            bytes_accessed=2 * N * C * HW * 4 + C * C * 4,
        ),
    )(xt, bt)
    return jnp.transpose(out.reshape(H, W, N, C), (2, 3, 0, 1))


# speedup vs baseline: 1.0012x; 1.0012x over previous
import jax
import jax.numpy as jnp
from jax.experimental import pallas as pl
from jax.experimental.pallas import tpu as pltpu


def _band_t(conv_weight: jnp.ndarray, channels: int, scale: float):
    k = conv_weight.shape[0]
    pad = (k - 1) // 2
    c = jnp.arange(channels)
    tap = c[:, None] - c[None, :] + pad
    valid = (tap >= 0) & (tap < k)
    w = jnp.where(valid, jnp.take(conv_weight, jnp.clip(tap, 0, k - 1)), 0.0)
    return (w * scale).astype(jnp.float32)


def _eca_body(x_ref, bt_ref, o_ref):
    x = x_ref[...]
    s = jnp.sum(x, axis=0)
    conv = jnp.dot(s, bt_ref[...],
                   preferred_element_type=jnp.float32)
    att = 1.0 / (1.0 + jnp.exp(-conv))
    o_ref[...] = x * att[None].astype(x.dtype)


def kernel(x, conv_weight):
    N, C, H, W = x.shape
    HW = H * W
    xt = jnp.transpose(x, (2, 3, 0, 1)).reshape(HW, N, C)
    bt = _band_t(conv_weight, C, 1.0 / float(HW))

    Bn = next((b for b in (32, 16, 8, 4, 2, 1) if N % b == 0), 1)

    out = pl.pallas_call(
        _eca_body,
        out_shape=jax.ShapeDtypeStruct((HW, N, C), x.dtype),
        grid_spec=pl.GridSpec(
            grid=(N // Bn,),
            in_specs=[pl.BlockSpec((HW, Bn, C), lambda i: (0, i, 0)),
                      pl.BlockSpec((C, C), lambda i: (0, 0))],
            out_specs=pl.BlockSpec((HW, Bn, C), lambda i: (0, i, 0)),
        ),
        compiler_params=pltpu.CompilerParams(
            dimension_semantics=("parallel",),
            vmem_limit_bytes=58 << 20,
        ),
        cost_estimate=pl.CostEstimate(
            flops=2 * N * C * HW + 2 * N * C * C,
            transcendentals=N * C,
            bytes_accessed=2 * N * C * HW * 4 + C * C * 4,
        ),
    )(xt, bt)
    return jnp.transpose(out.reshape(H, W, N, C), (2, 3, 0, 1))
